# Initial kernel scaffold; baseline (speedup 1.0000x reference)
import jax
import jax.numpy as jnp
from jax.experimental import pallas as pl


def kernel(x, edge_index, batch, params):
    raise NotImplementedError("write your pallas kernel here")


# speedup vs baseline: 9.7800x; 9.7800x over previous
import functools

import jax
import jax.numpy as jnp
from jax import lax
from jax.experimental import pallas as pl
from jax.experimental.pallas import tpu as pltpu
from jax.experimental.pallas import tpu_sc as plsc

NG = 64
D = 128
N = 10000
NP = 10240
NCH = NP // 128
E = 320000
NC, NS = 2, 16
NW = NC * NS
ECH = 79
EP = NW * ECH * 128
PAD_NODE = NP - 8
ROWS_PER_TILE = NP // NS


def _sc_agg_body(x_hbm, src_hbm, dst_hbm, zeros_hbm, out_hbm,
                 idx_s, idx_d, rows, aggr_sh, sem):
  cid = lax.axis_index("c")
  sid = lax.axis_index("s")
  r0 = sid * ROWS_PER_TILE
  pltpu.sync_copy(zeros_hbm.at[pl.ds(r0, ROWS_PER_TILE)],
                  aggr_sh.at[pl.ds(r0, ROWS_PER_TILE)])
  plsc.subcore_barrier()

  def chunk(j, carry):
    pltpu.sync_copy(src_hbm.at[cid, sid, j], idx_s)
    pltpu.sync_copy(dst_hbm.at[cid, sid, j], idx_d)
    pltpu.async_copy(x_hbm.at[idx_s], rows, sem).wait()
    pltpu.sync_copy(rows, aggr_sh.at[idx_d], add=True)
    return carry

  lax.fori_loop(0, ECH, chunk, 0)
  plsc.subcore_barrier()
  pltpu.sync_copy(aggr_sh.at[pl.ds(r0, ROWS_PER_TILE)],
                  out_hbm.at[cid, pl.ds(r0, ROWS_PER_TILE)])


@functools.cache
def _make_sc_agg():
  return pl.kernel(
      _sc_agg_body,
      out_type=jax.ShapeDtypeStruct((NC, NP, D), jnp.float32),
      mesh=plsc.VectorSubcoreMesh(core_axis_name="c", subcore_axis_name="s",
                                  num_cores=NC, num_subcores=NS),
      scratch_types=[
          pltpu.VMEM((128,), jnp.int32),
          pltpu.VMEM((128,), jnp.int32),
          pltpu.VMEM((128, D), jnp.float32),
          pltpu.VMEM_SHARED((NP, D), jnp.float32),
          pltpu.SemaphoreType.DMA,
      ],
  )


def _dense_body(a0, a1, xr, wrel, wroot, brel, gam, bet, zo):
  aggr = a0[...] + a1[...]
  acc = jnp.dot(aggr, wrel[...], preferred_element_type=jnp.float32)
  acc = acc + brel[...]
  acc = acc + jnp.dot(xr[...], wroot[...], preferred_element_type=jnp.float32)
  z = jnp.maximum(acc, 0.0)
  zo[...] = z / jnp.sqrt(jnp.float32(1.0 + 1e-5)) * gam[...] + bet[...]


_dense = pl.pallas_call(
    _dense_body,
    grid=(NCH,),
    in_specs=[
        pl.BlockSpec((128, D), lambda i: (i, 0)),
        pl.BlockSpec((128, D), lambda i: (i, 0)),
        pl.BlockSpec((128, D), lambda i: (i, 0)),
        pl.BlockSpec((D, D), lambda i: (0, 0)),
        pl.BlockSpec((D, D), lambda i: (0, 0)),
        pl.BlockSpec((1, D), lambda i: (0, 0)),
        pl.BlockSpec((1, D), lambda i: (0, 0)),
        pl.BlockSpec((1, D), lambda i: (0, 0)),
    ],
    out_specs=pl.BlockSpec((128, D), lambda i: (i, 0)),
    out_shape=jax.ShapeDtypeStruct((NP, D), jnp.float32),
)


def _nt(a, b):
  return lax.dot_general(a, b, (((1,), (1,)), ((), ())),
                         preferred_element_type=jnp.float32)


def _pool_body(z, valid, posv, batchv, onehot, onehotT, pvec, racc,
               glo, ghi, culo, cuhi,
               xo, valo, poso, racco, score_s):
  pv = pvec[...]
  nrm = jnp.sqrt(jnp.sum(pv * pv, axis=1, keepdims=True))

  def ph_a(c, carry):
    zc = z[pl.ds(c * 128, 128), :]
    s = _nt(pv, zc)
    score_s[pl.ds(c, 1), :] = jnp.tanh(s / nrm)
    return carry

  lax.fori_loop(0, NCH, ph_a, 0)

  def ph_b(c, acc):
    vrow = valid[pl.ds(c, 1), :]
    oc = onehot[pl.ds(c * 128, 128), :]
    return acc + jnp.dot(vrow, oc, preferred_element_type=jnp.float32)

  c64 = lax.fori_loop(0, NCH, ph_b, jnp.zeros((1, NG), jnp.float32))
  k64 = jnp.ceil(0.5 * c64)
  ii = lax.broadcasted_iota(jnp.int32, (NG, NG), 0)
  jj = lax.broadcasted_iota(jnp.int32, (NG, NG), 1)
  mlt = jnp.where(ii < jj, 1.0, 0.0).astype(jnp.float32)
  start64 = jnp.dot(c64, mlt, preferred_element_type=jnp.float32)

  i0 = lax.broadcasted_iota(jnp.int32, (128, 128), 0)
  i1 = lax.broadcasted_iota(jnp.int32, (128, 128), 1)
  ident = jnp.where(i0 == i1, 1.0, 0.0).astype(jnp.float32)

  def ph_c(c, carry):
    gapacc, cntacc = carry
    srow = score_s[pl.ds(c, 1), :]
    brow = batchv[pl.ds(c, 1), :]
    prow = posv[pl.ds(c, 1), :]
    vrow = valid[pl.ds(c, 1), :]

    def ph_u(u, acc):
      su = score_s[pl.ds(u, 1), :]
      bu = batchv[pl.ds(u, 1), :]
      pu = posv[pl.ds(u, 1), :]
      vu = valid[pl.ds(u, 1), :]
      su_c = _nt(ident, su)
      bu_c = _nt(ident, bu)
      pu_c = _nt(ident, pu)
      vu_c = _nt(ident, vu)
      beq = bu_c == brow
      sgt = su_c > srow
      seq = su_c == srow
      plt = pu_c < prow
      cond = beq & (vu_c > 0.5) & (sgt | (seq & plt))
      return acc + jnp.where(cond, 1.0, 0.0)

    acc_t = lax.fori_loop(culo[c], cuhi[c], ph_u,
                          jnp.zeros((128, 128), jnp.float32))
    rrow = jnp.sum(acc_t, axis=0, keepdims=True)
    oc = onehot[pl.ds(c * 128, 128), :]
    krow = _nt(k64, oc)
    strow = _nt(start64, oc)
    sel = jnp.where((rrow < krow) & (vrow > 0.5), 1.0, 0.0)
    valo[pl.ds(c, 1), :] = sel
    poso[pl.ds(c, 1), :] = strow + rrow
    ssel_c = _nt(ident, sel * srow)
    zc = z[pl.ds(c * 128, 128), :]
    xn = zc * ssel_c
    xo[pl.ds(c * 128, 128), :] = xn
    otc = onehotT[:, pl.ds(c * 128, 128)]
    gapacc = gapacc + jnp.dot(otc, xn, preferred_element_type=jnp.float32)
    cntacc = cntacc + jnp.dot(sel, oc, preferred_element_type=jnp.float32)
    return (gapacc, cntacc)

  gap_sum, cnt = lax.fori_loop(
      0, NCH, ph_c,
      (jnp.zeros((NG, D), jnp.float32), jnp.zeros((1, NG), jnp.float32)))

  gi = lax.broadcasted_iota(jnp.int32, (NG, NG), 0)
  gj = lax.broadcasted_iota(jnp.int32, (NG, NG), 1)
  ident_g = jnp.where(gi == gj, 1.0, 0.0).astype(jnp.float32)
  cnt_col = _nt(ident_g, cnt)
  gap = gap_sum / jnp.maximum(cnt_col, 1.0)

  neg_inf = jnp.float32(-jnp.inf)
  for g in range(NG):
    def ph_g(cc, m):
      xc = xo[pl.ds(cc * 128, 128), :]
      selrow = valo[pl.ds(cc, 1), :]
      brow = batchv[pl.ds(cc, 1), :]
      mrow = jnp.where((brow == jnp.float32(g)) & (selrow > 0.5), 1.0, 0.0)
      mcol = _nt(ident, mrow)
      xm = jnp.where(mcol > 0.5, xc, neg_inf)
      return jnp.maximum(m, jnp.max(xm, axis=0, keepdims=True))

    gmax = lax.fori_loop(glo[g], ghi[g], ph_g,
                         jnp.full((1, D), neg_inf, jnp.float32))
    racco[g:g + 1, 0:D] = racc[g:g + 1, 0:D] + gmax
    racco[g:g + 1, D:2 * D] = racc[g:g + 1, D:2 * D] + gap[g:g + 1, :]


_pool = pl.pallas_call(
    _pool_body,
    in_specs=[
        pl.BlockSpec(memory_space=pltpu.VMEM),
        pl.BlockSpec(memory_space=pltpu.VMEM),
        pl.BlockSpec(memory_space=pltpu.VMEM),
        pl.BlockSpec(memory_space=pltpu.VMEM),
        pl.BlockSpec(memory_space=pltpu.VMEM),
        pl.BlockSpec(memory_space=pltpu.VMEM),
        pl.BlockSpec(memory_space=pltpu.VMEM),
        pl.BlockSpec(memory_space=pltpu.VMEM),
        pl.BlockSpec(memory_space=pltpu.SMEM),
        pl.BlockSpec(memory_space=pltpu.SMEM),
        pl.BlockSpec(memory_space=pltpu.SMEM),
        pl.BlockSpec(memory_space=pltpu.SMEM),
    ],
    out_specs=[
        pl.BlockSpec(memory_space=pltpu.VMEM),
        pl.BlockSpec(memory_space=pltpu.VMEM),
        pl.BlockSpec(memory_space=pltpu.VMEM),
        pl.BlockSpec(memory_space=pltpu.VMEM),
    ],
    out_shape=[
        jax.ShapeDtypeStruct((NP, D), jnp.float32),
        jax.ShapeDtypeStruct((NCH, 128), jnp.float32),
        jax.ShapeDtypeStruct((NCH, 128), jnp.float32),
        jax.ShapeDtypeStruct((NG, 2 * D), jnp.float32),
    ],
    scratch_shapes=[pltpu.VMEM((NCH, 128), jnp.float32)],
)


def _mlp_body(r, w1, b1, w2, b2, w3, b3, out):
  h = jnp.dot(r[...], w1[...], preferred_element_type=jnp.float32) + b1[...]
  h = jnp.maximum(h, 0.0)
  h = jnp.dot(h, w2[...], preferred_element_type=jnp.float32) + b2[...]
  h = jnp.maximum(h, 0.0)
  lg = jnp.dot(h, w3[...], preferred_element_type=jnp.float32) + b3[...]
  m = jnp.max(lg, axis=1, keepdims=True)
  e = jnp.exp(lg - m)
  s = jnp.sum(e, axis=1, keepdims=True)
  out[...] = lg - m - jnp.log(s)


def _mlp(racc, p):
  return pl.pallas_call(
      _mlp_body,
      out_shape=jax.ShapeDtypeStruct((NG, 10), jnp.float32),
  )(racc, p["lin1_W"], p["lin1_b"].reshape(1, -1),
    p["lin2_W"], p["lin2_b"].reshape(1, -1),
    p["lin3_W"], p["lin3_b"].reshape(1, -1))


def _aggregate(xp, srcp, dstp, zeros_np):
  return _make_sc_agg()(xp, srcp, dstp, zeros_np)


@jax.jit
def _forward(x, edge_index, batch, params):
  f32 = jnp.float32
  xp = jnp.zeros((NP, D), f32).at[:N].set(x)
  batch_p = jnp.concatenate(
      [batch, jnp.full((NP - N,), NG - 1, jnp.int32)])
  bvf = batch_p.astype(f32).reshape(NCH, 128)
  valid = (jnp.arange(NP) < N).astype(f32).reshape(NCH, 128)
  pos = jnp.arange(NP, dtype=f32).reshape(NCH, 128)
  onehot = (batch_p[:, None] == jnp.arange(NG)[None, :]).astype(f32)
  onehot_t = onehot.T

  idxg = jnp.arange(NG, dtype=jnp.int32)
  lo = jnp.searchsorted(batch, idxg, side="left").astype(jnp.int32)
  hi = jnp.searchsorted(batch, idxg, side="right").astype(jnp.int32)
  glo = lo // 128
  ghi = jnp.where(hi > lo, (hi + 127) // 128, glo)
  cidx = jnp.arange(NCH)
  first_b = batch_p[cidx * 128]
  last_b = batch_p[cidx * 128 + 127]
  culo = lo[first_b] // 128
  cuhi = jnp.maximum((hi[last_b] + 127) // 128, culo).astype(jnp.int32)
  culo = culo.astype(jnp.int32)

  pad_e = jnp.full((EP - E,), PAD_NODE, jnp.int32)
  srcp = jnp.concatenate([edge_index[0], pad_e]).reshape(NC, NS, ECH, 128)
  dstp = jnp.concatenate([edge_index[1], pad_e]).reshape(NC, NS, ECH, 128)

  zeros_np = jnp.zeros((NP, D), f32)
  racc = jnp.zeros((NG, 2 * D), f32)

  for i in range(1, 6):
    agg = _aggregate(xp, srcp, dstp, zeros_np)
    z = _dense(agg[0], agg[1], xp,
               params[f"conv{i}_Wrel"], params[f"conv{i}_Wroot"],
               params[f"conv{i}_brel"].reshape(1, D),
               params[f"bn{i}_gamma"].reshape(1, D),
               params[f"bn{i}_beta"].reshape(1, D))
    xp, valid, pos, racc = _pool(
        z, valid, pos, bvf, onehot, onehot_t,
        params[f"pool{i}_p"].reshape(1, D), racc,
        glo, ghi, culo, cuhi)

  return _mlp(racc, params)


def kernel(x, edge_index, batch, params):
  return _forward(x, edge_index, batch, params)
